# baseline (speedup 1.0000x reference)
import functools

import numpy as np
import jax
import jax.numpy as jnp
from jax import lax
from jax.experimental import pallas as pl
from jax.experimental.pallas import tpu as pltpu
from jax.experimental.pallas import tpu_sc as plsc

_ROWS = 16384
_COLS = 200
_DIM = 20
_NC = 2
_NS = 16
_NW = _NC * _NS
_TJ = _COLS // 8
_TI = _ROWS // 128
_TI_W = _TI // _NW
_LANES = 16
_KGATHER = 10


def _body(lbl_hbm, table_hbm, tsplat_hbm, out_hbm,
          table_v, tsplat_v, lA, lB, r0, r1, sA, sB, s0, s1):
    wid = lax.axis_index("s") * _NC + lax.axis_index("c")
    ti0 = wid * _TI_W

    pltpu.sync_copy(table_hbm, table_v)
    pltpu.sync_copy(tsplat_hbm, tsplat_v)

    def lbl_slab(tj):
        return lbl_hbm.at[tj, pl.ds(ti0, _TI_W), :, :]

    def start_lbl(tj, lbuf, sem):
        pltpu.async_copy(lbl_slab(tj), lbuf, sem)

    def wait_lbl(lbuf, sem):
        pltpu.make_async_copy(lbl_slab(0), lbuf, sem).wait()

    splats = [[tsplat_v[r * _DIM + k, :]
               for k in range(_KGATHER, _DIM)] for r in range(3)]

    def compute_step(q, lbuf, rbuf):
        @plsc.parallel_loop(0, 128, unroll=1)
        def chunk(p):
            sj = lax.shift_right_logical(p, 4)
            c = lax.bitwise_and(p, 15)
            tix = lax.shift_right_logical(c, 3)
            cm8 = lax.bitwise_and(c, 7)
            lbl = lbuf[2 * q + tix, sj, pl.ds(cm8 * _LANES, _LANES)]
            g0 = lbl * _DIM
            for k in range(_KGATHER):
                vals = plsc.load_gather(table_v, [g0 + k])
                rbuf[k, tix, sj, pl.ds(cm8 * _LANES, _LANES)] = vals
            m1 = lbl == 1
            m2 = lbl == 2
            for k in range(_KGATHER, _DIM):
                t0, t1, t2 = (splats[r][k - _KGATHER] for r in range(3))
                vals = jnp.where(m2, t2, jnp.where(m1, t1, t0))
                rbuf[k, tix, sj, pl.ds(cm8 * _LANES, _LANES)] = vals

    def start_out(tj, q, rbuf, sem):
        return pltpu.async_copy(
            rbuf, out_hbm.at[:, tj, pl.ds(ti0 + 2 * q, 2), :, :], sem)

    def drain_out(rbuf, sem):
        pltpu.make_async_copy(
            rbuf, out_hbm.at[:, 0, pl.ds(0, 2), :, :], sem).wait()

    def tj_work(tj, lbuf, sem, nbuf, nsem, first):
        wait_lbl(lbuf, sem)
        start_lbl(jnp.minimum(tj + 1, _TJ - 1), nbuf, nsem)
        if not first:
            drain_out(r0, s0)
        compute_step(0, lbuf, r0)
        start_out(tj, 0, r0, s0)
        if not first:
            drain_out(r1, s1)
        compute_step(1, lbuf, r1)
        start_out(tj, 1, r1, s1)

    start_lbl(0, lA, sA)
    tj_work(0, lA, sA, lB, sB, first=True)

    def pair(g, carry):
        tj_work(2 * g + 1, lB, sB, lA, sA, first=False)
        tj_work(2 * g + 2, lA, sA, lB, sB, first=False)
        return carry

    lax.fori_loop(0, (_TJ - 1) // 2, pair, 0)

    wait_lbl(lB, sB)
    drain_out(r0, s0)
    drain_out(r1, s1)


def kernel(speaker_labels, table):
    lbl_t = (speaker_labels.T
             .reshape(_TJ, 8, _TI, 128)
             .transpose(0, 2, 1, 3))
    grid_kernel = pl.kernel(
        _body,
        out_type=jax.ShapeDtypeStruct((_DIM, _TJ, _TI, 8, 128), jnp.float32),
        mesh=plsc.VectorSubcoreMesh(
            core_axis_name="c", subcore_axis_name="s",
            num_cores=_NC, num_subcores=_NS,
        ),
        scratch_types=[
            pltpu.VMEM((3 * _DIM,), jnp.float32),
            pltpu.VMEM((3 * _DIM, _LANES), jnp.float32),
            pltpu.VMEM((_TI_W, 8, 128), jnp.int32),
            pltpu.VMEM((_TI_W, 8, 128), jnp.int32),
            pltpu.VMEM((_DIM, 2, 8, 128), jnp.float32),
            pltpu.VMEM((_DIM, 2, 8, 128), jnp.float32),
            pltpu.SemaphoreType.DMA,
            pltpu.SemaphoreType.DMA,
            pltpu.SemaphoreType.DMA,
            pltpu.SemaphoreType.DMA,
        ],
        compiler_params=pltpu.CompilerParams(
            use_tc_tiling_on_sc=False, needs_layout_passes=False),
    )
    tflat = table.reshape(3 * _DIM)
    tsplat = jnp.broadcast_to(tflat[:, None], (3 * _DIM, _LANES))
    out_t = grid_kernel(lbl_t, tflat, tsplat)
    return out_t.transpose(2, 4, 1, 3, 0).reshape(_ROWS, _COLS, _DIM)


# speedup vs baseline: 1.6708x; 1.6708x over previous
import functools

import numpy as np
import jax
import jax.numpy as jnp
from jax import lax
from jax.experimental import pallas as pl
from jax.experimental.pallas import tpu as pltpu
from jax.experimental.pallas import tpu_sc as plsc

_ROWS = 16384
_COLS = 200
_DIM = 20
_NC = 2
_NS = 16
_NW = _NC * _NS
_TJ = _COLS // 8
_TI = _ROWS // 128
_TI_W = _TI // _NW
_LANES = 16


def _body(lbl_hbm, table_hbm, out_hbm,
          table_v, lA, lB, r0, r1, sA, sB, s0, s1):
    wid = lax.axis_index("s") * _NC + lax.axis_index("c")
    ti0 = wid * _TI_W

    pltpu.sync_copy(table_hbm, table_v)

    def lbl_slab(tj):
        return lbl_hbm.at[tj, pl.ds(ti0, _TI_W), :, :]

    def start_lbl(tj, lbuf, sem):
        pltpu.async_copy(lbl_slab(tj), lbuf, sem)

    def wait_lbl(lbuf, sem):
        pltpu.make_async_copy(lbl_slab(0), lbuf, sem).wait()

    def compute_step(q, lbuf, rbuf):
        @plsc.parallel_loop(0, 128, unroll=1)
        def chunk(p):
            sj = lax.shift_right_logical(p, 4)
            c = lax.bitwise_and(p, 15)
            tix = lax.shift_right_logical(c, 3)
            cm8 = lax.bitwise_and(c, 7)
            lbl = lbuf[2 * q + tix, sj, pl.ds(cm8 * _LANES, _LANES)]
            g0 = lbl * _DIM
            for k in range(_DIM):
                vals = plsc.load_gather(table_v, [g0 + k])
                rbuf[k, tix, sj, pl.ds(cm8 * _LANES, _LANES)] = vals

    def start_out(tj, q, rbuf, sem):
        return pltpu.async_copy(
            rbuf, out_hbm.at[:, tj, pl.ds(ti0 + 2 * q, 2), :, :], sem)

    def drain_out(rbuf, sem):
        pltpu.make_async_copy(
            rbuf, out_hbm.at[:, 0, pl.ds(0, 2), :, :], sem).wait()

    def tj_work(tj, lbuf, sem, nbuf, nsem, first):
        wait_lbl(lbuf, sem)
        start_lbl(jnp.minimum(tj + 1, _TJ - 1), nbuf, nsem)
        if not first:
            drain_out(r0, s0)
        compute_step(0, lbuf, r0)
        start_out(tj, 0, r0, s0)
        if not first:
            drain_out(r1, s1)
        compute_step(1, lbuf, r1)
        start_out(tj, 1, r1, s1)

    start_lbl(0, lA, sA)
    tj_work(0, lA, sA, lB, sB, first=True)

    def pair(g, carry):
        tj_work(2 * g + 1, lB, sB, lA, sA, first=False)
        tj_work(2 * g + 2, lA, sA, lB, sB, first=False)
        return carry

    lax.fori_loop(0, (_TJ - 1) // 2, pair, 0)

    wait_lbl(lB, sB)
    drain_out(r0, s0)
    drain_out(r1, s1)


def kernel(speaker_labels, table):
    lbl_t = (speaker_labels.T
             .reshape(_TJ, 8, _TI, 128)
             .transpose(0, 2, 1, 3))
    grid_kernel = pl.kernel(
        _body,
        out_type=jax.ShapeDtypeStruct((_DIM, _TJ, _TI, 8, 128), jnp.float32),
        mesh=plsc.VectorSubcoreMesh(
            core_axis_name="c", subcore_axis_name="s",
            num_cores=_NC, num_subcores=_NS,
        ),
        scratch_types=[
            pltpu.VMEM((3 * _DIM,), jnp.float32),
            pltpu.VMEM((_TI_W, 8, 128), jnp.int32),
            pltpu.VMEM((_TI_W, 8, 128), jnp.int32),
            pltpu.VMEM((_DIM, 2, 8, 128), jnp.float32),
            pltpu.VMEM((_DIM, 2, 8, 128), jnp.float32),
            pltpu.SemaphoreType.DMA,
            pltpu.SemaphoreType.DMA,
            pltpu.SemaphoreType.DMA,
            pltpu.SemaphoreType.DMA,
        ],
        compiler_params=pltpu.CompilerParams(
            use_tc_tiling_on_sc=False, needs_layout_passes=False),
    )
    out_t = grid_kernel(lbl_t, table.reshape(3 * _DIM))
    return out_t.transpose(2, 4, 1, 3, 0).reshape(_ROWS, _COLS, _DIM)
